# baseline (speedup 1.0000x reference)
import jax
import jax.numpy as jnp
import numpy as np
from jax import lax
from jax.experimental import pallas as pl
from jax.experimental.pallas import tpu as pltpu

L = 2048
DM = 1024
H = 16
DH = 64
DFF = 2048
U = 64
SK = 128
SCALE = 1.0 / np.sqrt(DH)
RT = 256
HI = lax.Precision.HIGHEST

_SIDX = np.asarray(jax.random.permutation(jax.random.key(42), L))[:SK]
_SEL = np.zeros((SK, L), np.float32)
_SEL[np.arange(SK), _SIDX] = 1.0


def _qkv_body(x_ref, wq_ref, wk_ref, wv_ref, bq_ref, bk_ref, bv_ref,
              q_ref, k_ref, v_ref):
    xx = x_ref[...]
    q_ref[...] = jnp.dot(xx, wq_ref[...], precision=HI,
                         preferred_element_type=jnp.float32) + bq_ref[...]
    k_ref[...] = jnp.dot(xx, wk_ref[...], precision=HI,
                         preferred_element_type=jnp.float32) + bk_ref[...]
    v_ref[...] = jnp.dot(xx, wv_ref[...], precision=HI,
                         preferred_element_type=jnp.float32) + bv_ref[...]


def _m_body(q_ref, k_ref, sel_ref, m_ref):
    ks = jnp.dot(sel_ref[...], k_ref[...], precision=HI,
                 preferred_element_type=jnp.float32)
    for h in range(H):
        qh = q_ref[:, h * DH:(h + 1) * DH]
        ksh = ks[:, h * DH:(h + 1) * DH]
        ssT = lax.dot_general(ksh, qh, (((1,), (1,)), ((), ())),
                              precision=HI,
                              preferred_element_type=jnp.float32) * SCALE
        m_ref[h:h + 1, :] = (jnp.max(ssT, axis=0, keepdims=True)
                             - jnp.mean(ssT, axis=0, keepdims=True))


def _attn_body(q_ref, k_ref, v_ref, mr_ref, mc_ref, wo_ref, bo_ref, out_ref):
    acc = jnp.broadcast_to(bo_ref[...], (L, DM))
    for h in range(H):
        mr = mr_ref[h:h + 1, :]
        pts = []
        masks = []
        for c in range(L // RT):
            mc = mc_ref[c * RT:(c + 1) * RT, h:h + 1]
            colid = lax.broadcasted_iota(jnp.int32, (RT, L), 1)
            rowid = lax.broadcasted_iota(jnp.int32, (RT, L), 0) + c * RT
            cmp = (mr > mc) | ((mr == mc) & (colid < rowid))
            rank = jnp.sum(cmp.astype(jnp.int32), axis=1, keepdims=True)
            uio = lax.broadcasted_iota(jnp.int32, (RT, U), 1)
            pts.append((rank == uio).astype(jnp.float32))
            masks.append((rank < U).astype(jnp.float32))
        pt = jnp.concatenate(pts, axis=0)
        maskf = jnp.concatenate(masks, axis=0)

        qh = q_ref[:, h * DH:(h + 1) * DH]
        kh = k_ref[:, h * DH:(h + 1) * DH]
        vh = v_ref[:, h * DH:(h + 1) * DH]
        qtop = lax.dot_general(pt, qh, (((0,), (0,)), ((), ())),
                               precision=HI,
                               preferred_element_type=jnp.float32)
        s = lax.dot_general(qtop, kh, (((1,), (1,)), ((), ())),
                            precision=HI,
                            preferred_element_type=jnp.float32) * SCALE
        smax = jnp.max(s, axis=1, keepdims=True)
        e = jnp.exp(s - smax)
        a = e / jnp.sum(e, axis=1, keepdims=True)
        otop = jnp.dot(a, vh, precision=HI,
                       preferred_element_type=jnp.float32)
        vmean = jnp.mean(vh, axis=0, keepdims=True)
        ctx = (jnp.dot(pt, otop, precision=HI,
                       preferred_element_type=jnp.float32)
               + (1.0 - maskf) * vmean)
        acc = acc + jnp.dot(ctx, wo_ref[h * DH:(h + 1) * DH, :],
                            precision=HI,
                            preferred_element_type=jnp.float32)
    out_ref[...] = acc


def _ln(y, g, b):
    mu = jnp.mean(y, axis=1, keepdims=True)
    var = jnp.mean((y - mu) * (y - mu), axis=1, keepdims=True)
    return (y - mu) / jnp.sqrt(var + 1e-6) * g + b


def _ffn_body(x_ref, attn_ref, g1_ref, b1n_ref, w1_ref, b1_ref, w2_ref,
              b2_ref, g2_ref, b2n_ref, out_ref):
    h1 = _ln(x_ref[...] + attn_ref[...], g1_ref[...], b1n_ref[...])
    t = jnp.maximum(jnp.dot(h1, w1_ref[...], precision=HI,
                            preferred_element_type=jnp.float32)
                    + b1_ref[...], 0.0)
    f = jnp.dot(t, w2_ref[...], precision=HI,
                preferred_element_type=jnp.float32) + b2_ref[...]
    out_ref[...] = _ln(h1 + f, g2_ref[...], b2n_ref[...])


def kernel(x, Wq, bq, Wk, bk, Wv, bv, Wo, bo, ln1_g, ln1_b,
           W1, b1, W2, b2, ln2_g, ln2_b):
    x2 = x.reshape(L, DM)
    row = lambda t: t.reshape(1, -1)
    nrt = L // RT

    q, k, v = pl.pallas_call(
        _qkv_body,
        grid=(nrt,),
        in_specs=[
            pl.BlockSpec((RT, DM), lambda i: (i, 0)),
            pl.BlockSpec((DM, DM), lambda i: (0, 0)),
            pl.BlockSpec((DM, DM), lambda i: (0, 0)),
            pl.BlockSpec((DM, DM), lambda i: (0, 0)),
            pl.BlockSpec((1, DM), lambda i: (0, 0)),
            pl.BlockSpec((1, DM), lambda i: (0, 0)),
            pl.BlockSpec((1, DM), lambda i: (0, 0)),
        ],
        out_specs=[
            pl.BlockSpec((RT, DM), lambda i: (i, 0)),
            pl.BlockSpec((RT, DM), lambda i: (i, 0)),
            pl.BlockSpec((RT, DM), lambda i: (i, 0)),
        ],
        out_shape=[jax.ShapeDtypeStruct((L, DM), jnp.float32)] * 3,
    )(x2, Wq, Wk, Wv, row(bq), row(bk), row(bv))

    m_all = pl.pallas_call(
        _m_body,
        out_shape=jax.ShapeDtypeStruct((H, L), jnp.float32),
    )(q, k, jnp.asarray(_SEL))

    m_t = m_all.T

    attn = pl.pallas_call(
        _attn_body,
        out_shape=jax.ShapeDtypeStruct((L, DM), jnp.float32),
    )(q, k, v, m_all, m_t, Wo, row(bo))

    out = pl.pallas_call(
        _ffn_body,
        grid=(nrt,),
        in_specs=[
            pl.BlockSpec((RT, DM), lambda i: (i, 0)),
            pl.BlockSpec((RT, DM), lambda i: (i, 0)),
            pl.BlockSpec((1, DM), lambda i: (0, 0)),
            pl.BlockSpec((1, DM), lambda i: (0, 0)),
            pl.BlockSpec((DM, DFF), lambda i: (0, 0)),
            pl.BlockSpec((1, DFF), lambda i: (0, 0)),
            pl.BlockSpec((DFF, DM), lambda i: (0, 0)),
            pl.BlockSpec((1, DM), lambda i: (0, 0)),
            pl.BlockSpec((1, DM), lambda i: (0, 0)),
            pl.BlockSpec((1, DM), lambda i: (0, 0)),
        ],
        out_specs=pl.BlockSpec((RT, DM), lambda i: (i, 0)),
        out_shape=jax.ShapeDtypeStruct((L, DM), jnp.float32),
    )(x2, attn, row(ln1_g), row(ln1_b), W1, row(b1), W2, row(b2),
      row(ln2_g), row(ln2_b))

    return out.reshape(1, L, DM)


# speedup vs baseline: 1.3341x; 1.3341x over previous
import jax
import jax.numpy as jnp
import numpy as np
from jax import lax
from jax.experimental import pallas as pl
from jax.experimental.pallas import tpu as pltpu

L = 2048
DM = 1024
H = 16
DH = 64
DFF = 2048
U = 64
SK = 128
SCALE = 1.0 / np.sqrt(DH)
RT = 256
HI = lax.Precision.HIGHEST
F32 = jnp.float32
BF = jnp.bfloat16


def _sel_matrix():
    sidx = jax.random.permutation(jax.random.key(42), L)[:SK]
    return jax.nn.one_hot(sidx, L, dtype=F32)


def _dot(a, b, dims=None):
    if dims is None:
        dims = (((1,), (0,)), ((), ()))
    prec = lax.Precision.DEFAULT if a.dtype == BF else HI
    return lax.dot_general(a, b, dims, precision=prec,
                           preferred_element_type=F32)


def _heads(t):
    return jnp.transpose(t.reshape(RT, H, DH), (1, 0, 2))


def _qkv_body(x_ref, wq_ref, wk_ref, wv_bf_ref, bq_ref, bk_ref, bv_ref,
              q_ref, k_ref, v_ref):
    xx = x_ref[...]
    q_ref[...] = _heads(_dot(xx, wq_ref[...]) + bq_ref[...])
    k_ref[...] = _heads(_dot(xx, wk_ref[...]) + bk_ref[...])
    v_ref[...] = _heads(_dot(xx.astype(BF), wv_bf_ref[...])
                        + bv_ref[...]).astype(BF)


def _m_body(q_ref, k_ref, sel_ref, m_ref):
    qh = q_ref[0]
    kh = k_ref[0]
    ksh = _dot(sel_ref[...], kh)
    ssT = _dot(ksh, qh, (((1,), (1,)), ((), ()))) * SCALE
    m_ref[0] = (jnp.max(ssT, axis=0, keepdims=True)
                - jnp.mean(ssT, axis=0, keepdims=True))


def _sel_body(mr_ref, mc_ref, pt_ref):
    uio = lax.broadcasted_iota(jnp.int32, (RT, U), 1)

    def chunk_body(c, carry):
        base = c * RT
        colid = lax.broadcasted_iota(jnp.int32, (RT, L), 1)
        rowid = lax.broadcasted_iota(jnp.int32, (RT, L), 0) + base
        lt = colid < rowid
        for h in range(H):
            mr = mr_ref[h:h + 1, :]
            mc = mc_ref[pl.ds(base, RT), h:h + 1]
            cmp = (mr > mc) | ((mr == mc) & lt)
            rank = jnp.sum(cmp.astype(jnp.int32), axis=1, keepdims=True)
            pt_ref[h, pl.ds(base, RT), :] = (rank == uio).astype(BF)
        return carry

    lax.fori_loop(0, L // RT, chunk_body, 0)


def _attn_body(q_ref, k_ref, v_ref, pt_ref, ctx_ref):
    pt = pt_ref[0].astype(F32)
    qh = q_ref[0]
    kh = k_ref[0]
    vh_bf = v_ref[0]
    qtop = _dot(pt, qh, (((0,), (0,)), ((), ())))
    s = _dot(qtop, kh, (((1,), (1,)), ((), ()))) * SCALE
    smax = jnp.max(s, axis=1, keepdims=True)
    e = jnp.exp(s - smax)
    a = e / jnp.sum(e, axis=1, keepdims=True)
    otop = _dot(a.astype(BF), vh_bf)
    vmean = jnp.mean(vh_bf.astype(F32), axis=0, keepdims=True)
    notsel = 1.0 - jnp.sum(pt, axis=1, keepdims=True)
    ctx_ref[0] = _dot(pt, otop) + notsel * vmean


def _ln(y, g, b):
    mu = jnp.mean(y, axis=1, keepdims=True)
    var = jnp.mean((y - mu) * (y - mu), axis=1, keepdims=True)
    return (y - mu) / jnp.sqrt(var + 1e-6) * g + b


def _ffn_body(x_ref, ctx_ref, wo_bf_ref, bo_ref, g1_ref, b1n_ref,
              w1_bf_ref, b1_ref, w2_bf_ref, b2_ref, g2_ref, b2n_ref,
              out_ref):
    ctx = jnp.transpose(ctx_ref[...], (1, 0, 2)).reshape(RT, DM)
    attn = _dot(ctx.astype(BF), wo_bf_ref[...]) + bo_ref[...]
    h1 = _ln(x_ref[...] + attn, g1_ref[...], b1n_ref[...])
    t = jnp.maximum(_dot(h1.astype(BF), w1_bf_ref[...]) + b1_ref[...], 0.0)
    f = _dot(t.astype(BF), w2_bf_ref[...]) + b2_ref[...]
    out_ref[...] = _ln(h1 + f, g2_ref[...], b2n_ref[...])


def kernel(x, Wq, bq, Wk, bk, Wv, bv, Wo, bo, ln1_g, ln1_b,
           W1, b1, W2, b2, ln2_g, ln2_b):
    x2 = x.reshape(L, DM)
    row = lambda t: t.reshape(1, -1)
    nrt = L // RT

    q, k, v = pl.pallas_call(
        _qkv_body,
        grid=(nrt,),
        in_specs=[
            pl.BlockSpec((RT, DM), lambda i: (i, 0)),
            pl.BlockSpec((DM, DM), lambda i: (0, 0)),
            pl.BlockSpec((DM, DM), lambda i: (0, 0)),
            pl.BlockSpec((DM, DM), lambda i: (0, 0)),
            pl.BlockSpec((1, DM), lambda i: (0, 0)),
            pl.BlockSpec((1, DM), lambda i: (0, 0)),
            pl.BlockSpec((1, DM), lambda i: (0, 0)),
        ],
        out_specs=[
            pl.BlockSpec((H, RT, DH), lambda i: (0, i, 0)),
            pl.BlockSpec((H, RT, DH), lambda i: (0, i, 0)),
            pl.BlockSpec((H, RT, DH), lambda i: (0, i, 0)),
        ],
        out_shape=[
            jax.ShapeDtypeStruct((H, L, DH), F32),
            jax.ShapeDtypeStruct((H, L, DH), F32),
            jax.ShapeDtypeStruct((H, L, DH), BF),
        ],
    )(x2, Wq, Wk, Wv.astype(BF), row(bq), row(bk), row(bv))

    m3 = pl.pallas_call(
        _m_body,
        grid=(H,),
        in_specs=[
            pl.BlockSpec((1, L, DH), lambda h: (h, 0, 0)),
            pl.BlockSpec((1, L, DH), lambda h: (h, 0, 0)),
            pl.BlockSpec((SK, L), lambda h: (0, 0)),
        ],
        out_specs=pl.BlockSpec((1, 1, L), lambda h: (h, 0, 0)),
        out_shape=jax.ShapeDtypeStruct((H, 1, L), F32),
    )(q, k, _sel_matrix())

    m_all = m3.reshape(H, L)
    m_t = m_all.T

    pt_all = pl.pallas_call(
        _sel_body,
        out_shape=jax.ShapeDtypeStruct((H, L, U), BF),
    )(m_all, m_t)

    ctx = pl.pallas_call(
        _attn_body,
        grid=(H,),
        in_specs=[
            pl.BlockSpec((1, L, DH), lambda h: (h, 0, 0)),
            pl.BlockSpec((1, L, DH), lambda h: (h, 0, 0)),
            pl.BlockSpec((1, L, DH), lambda h: (h, 0, 0)),
            pl.BlockSpec((1, L, U), lambda h: (h, 0, 0)),
        ],
        out_specs=pl.BlockSpec((1, L, DH), lambda h: (h, 0, 0)),
        out_shape=jax.ShapeDtypeStruct((H, L, DH), F32),
    )(q, k, v, pt_all)

    out = pl.pallas_call(
        _ffn_body,
        grid=(nrt,),
        in_specs=[
            pl.BlockSpec((RT, DM), lambda i: (i, 0)),
            pl.BlockSpec((H, RT, DH), lambda i: (0, i, 0)),
            pl.BlockSpec((DM, DM), lambda i: (0, 0)),
            pl.BlockSpec((1, DM), lambda i: (0, 0)),
            pl.BlockSpec((1, DM), lambda i: (0, 0)),
            pl.BlockSpec((1, DM), lambda i: (0, 0)),
            pl.BlockSpec((DM, DFF), lambda i: (0, 0)),
            pl.BlockSpec((1, DFF), lambda i: (0, 0)),
            pl.BlockSpec((DFF, DM), lambda i: (0, 0)),
            pl.BlockSpec((1, DM), lambda i: (0, 0)),
            pl.BlockSpec((1, DM), lambda i: (0, 0)),
            pl.BlockSpec((1, DM), lambda i: (0, 0)),
        ],
        out_specs=pl.BlockSpec((RT, DM), lambda i: (i, 0)),
        out_shape=jax.ShapeDtypeStruct((L, DM), F32),
    )(x2, ctx, Wo.astype(BF), row(bo), row(ln1_g), row(ln1_b),
      W1.astype(BF), row(b1), W2.astype(BF), row(b2), row(ln2_g),
      row(ln2_b))

    return out.reshape(1, L, DM)


# speedup vs baseline: 2.0244x; 1.5174x over previous
import jax
import jax.numpy as jnp
import numpy as np
from jax import lax
from jax.experimental import pallas as pl
from jax.experimental.pallas import tpu as pltpu

L = 2048
DM = 1024
H = 16
DH = 64
DFF = 2048
U = 64
SK = 128
SCALE = 1.0 / np.sqrt(DH)
RT = 256
F32 = jnp.float32
BF = jnp.bfloat16


def _sel_matrix():
    sidx = jax.random.permutation(jax.random.key(42), L)[:SK]
    return jax.nn.one_hot(sidx, L, dtype=F32)


def _dot(a, b, dims=None):
    if dims is None:
        dims = (((1,), (0,)), ((), ()))
    return lax.dot_general(a, b, dims, precision=lax.Precision.DEFAULT,
                           preferred_element_type=F32)


def _qkv_body(x_ref, wq_ref, wk_ref, wv_bf_ref, bq_ref, bk_ref, bv_ref,
              q_ref, k_ref, v_ref):
    xx = x_ref[...]
    q_ref[...] = _dot(xx, wq_ref[...]) + bq_ref[...]
    k_ref[...] = _dot(xx, wk_ref[...]) + bk_ref[...]
    v_ref[...] = (_dot(xx.astype(BF), wv_bf_ref[...])
                  + bv_ref[...]).astype(BF)


def _m_body(q_ref, k_ref, sel_ref, m_ref):
    ks2 = _dot(sel_ref[...], k_ref[...])
    for hh in range(2):
        qh = q_ref[:, hh * DH:(hh + 1) * DH]
        ksh = ks2[:, hh * DH:(hh + 1) * DH]
        ssT = _dot(ksh, qh, (((1,), (1,)), ((), ()))) * SCALE
        m_ref[hh, 0, :] = (jnp.max(ssT, axis=0)
                           - jnp.mean(ssT, axis=0))


def _sel_body(mr_ref, mc_ref, pt_ref):
    uio = lax.broadcasted_iota(jnp.int32, (RT, U), 1)

    def chunk_body(c, carry):
        base = c * RT
        colid = lax.broadcasted_iota(jnp.int32, (RT, L), 1)
        rowid = lax.broadcasted_iota(jnp.int32, (RT, L), 0) + base
        lt = colid < rowid
        for h in range(H):
            mr = mr_ref[h:h + 1, :]
            mc = mc_ref[pl.ds(base, RT), h:h + 1]
            cmp = (mr > mc) | ((mr == mc) & lt)
            rank = jnp.sum(cmp.astype(jnp.int32), axis=1, keepdims=True)
            pt_ref[h, pl.ds(base, RT), :] = (rank == uio).astype(BF)
        return carry

    lax.fori_loop(0, L // RT, chunk_body, 0)


def _attn_body(q_ref, k_ref, v_ref, pt_ref, ctx_ref):
    for hh in range(2):
        cs, ce = hh * DH, (hh + 1) * DH
        pt = pt_ref[hh].astype(F32)
        qh = q_ref[:, cs:ce]
        kh = k_ref[:, cs:ce]
        vh_bf = v_ref[:, cs:ce]
        qtop = _dot(pt, qh, (((0,), (0,)), ((), ())))
        s = _dot(qtop, kh, (((1,), (1,)), ((), ()))) * SCALE
        smax = jnp.max(s, axis=1, keepdims=True)
        e = jnp.exp(s - smax)
        a = e / jnp.sum(e, axis=1, keepdims=True)
        otop = _dot(a.astype(BF), vh_bf)
        vmean = jnp.mean(vh_bf.astype(F32), axis=0, keepdims=True)
        notsel = 1.0 - jnp.sum(pt, axis=1, keepdims=True)
        ctx_ref[:, cs:ce] = _dot(pt, otop) + notsel * vmean


def _ln(y, g, b):
    mu = jnp.mean(y, axis=1, keepdims=True)
    var = jnp.mean((y - mu) * (y - mu), axis=1, keepdims=True)
    return (y - mu) / jnp.sqrt(var + 1e-6) * g + b


def _ffn_body(x_ref, ctx_ref, wo_bf_ref, bo_ref, g1_ref, b1n_ref,
              w1_bf_ref, b1_ref, w2_bf_ref, b2_ref, g2_ref, b2n_ref,
              out_ref):
    attn = _dot(ctx_ref[...].astype(BF), wo_bf_ref[...]) + bo_ref[...]
    h1 = _ln(x_ref[...] + attn, g1_ref[...], b1n_ref[...])
    t = jnp.maximum(_dot(h1.astype(BF), w1_bf_ref[...]) + b1_ref[...], 0.0)
    f = _dot(t.astype(BF), w2_bf_ref[...]) + b2_ref[...]
    out_ref[...] = _ln(h1 + f, g2_ref[...], b2n_ref[...])


def kernel(x, Wq, bq, Wk, bk, Wv, bv, Wo, bo, ln1_g, ln1_b,
           W1, b1, W2, b2, ln2_g, ln2_b):
    x2 = x.reshape(L, DM)
    row = lambda t: t.reshape(1, -1)
    nrt = L // RT

    q, k, v = pl.pallas_call(
        _qkv_body,
        grid=(nrt,),
        in_specs=[
            pl.BlockSpec((RT, DM), lambda i: (i, 0)),
            pl.BlockSpec((DM, DM), lambda i: (0, 0)),
            pl.BlockSpec((DM, DM), lambda i: (0, 0)),
            pl.BlockSpec((DM, DM), lambda i: (0, 0)),
            pl.BlockSpec((1, DM), lambda i: (0, 0)),
            pl.BlockSpec((1, DM), lambda i: (0, 0)),
            pl.BlockSpec((1, DM), lambda i: (0, 0)),
        ],
        out_specs=[
            pl.BlockSpec((RT, DM), lambda i: (i, 0)),
            pl.BlockSpec((RT, DM), lambda i: (i, 0)),
            pl.BlockSpec((RT, DM), lambda i: (i, 0)),
        ],
        out_shape=[
            jax.ShapeDtypeStruct((L, DM), F32),
            jax.ShapeDtypeStruct((L, DM), F32),
            jax.ShapeDtypeStruct((L, DM), BF),
        ],
    )(x2, Wq, Wk, Wv.astype(BF), row(bq), row(bk), row(bv))

    m3 = pl.pallas_call(
        _m_body,
        grid=(H // 2,),
        in_specs=[
            pl.BlockSpec((L, 2 * DH), lambda g: (0, g)),
            pl.BlockSpec((L, 2 * DH), lambda g: (0, g)),
            pl.BlockSpec((SK, L), lambda g: (0, 0)),
        ],
        out_specs=pl.BlockSpec((2, 1, L), lambda g: (g, 0, 0)),
        out_shape=jax.ShapeDtypeStruct((H, 1, L), F32),
    )(q, k, _sel_matrix())

    m_all = m3.reshape(H, L)
    m_t = m_all.T

    pt_all = pl.pallas_call(
        _sel_body,
        out_shape=jax.ShapeDtypeStruct((H, L, U), BF),
    )(m_all, m_t)

    ctx = pl.pallas_call(
        _attn_body,
        grid=(H // 2,),
        in_specs=[
            pl.BlockSpec((L, 2 * DH), lambda g: (0, g)),
            pl.BlockSpec((L, 2 * DH), lambda g: (0, g)),
            pl.BlockSpec((L, 2 * DH), lambda g: (0, g)),
            pl.BlockSpec((2, L, U), lambda g: (g, 0, 0)),
        ],
        out_specs=pl.BlockSpec((L, 2 * DH), lambda g: (0, g)),
        out_shape=jax.ShapeDtypeStruct((L, DM), F32),
    )(q, k, v, pt_all)

    out = pl.pallas_call(
        _ffn_body,
        grid=(nrt,),
        in_specs=[
            pl.BlockSpec((RT, DM), lambda i: (i, 0)),
            pl.BlockSpec((RT, DM), lambda i: (i, 0)),
            pl.BlockSpec((DM, DM), lambda i: (0, 0)),
            pl.BlockSpec((1, DM), lambda i: (0, 0)),
            pl.BlockSpec((1, DM), lambda i: (0, 0)),
            pl.BlockSpec((1, DM), lambda i: (0, 0)),
            pl.BlockSpec((DM, DFF), lambda i: (0, 0)),
            pl.BlockSpec((1, DFF), lambda i: (0, 0)),
            pl.BlockSpec((DFF, DM), lambda i: (0, 0)),
            pl.BlockSpec((1, DM), lambda i: (0, 0)),
            pl.BlockSpec((1, DM), lambda i: (0, 0)),
            pl.BlockSpec((1, DM), lambda i: (0, 0)),
        ],
        out_specs=pl.BlockSpec((RT, DM), lambda i: (i, 0)),
        out_shape=jax.ShapeDtypeStruct((L, DM), F32),
    )(x2, ctx, Wo.astype(BF), row(bo), row(ln1_g), row(ln1_b),
      W1.astype(BF), row(b1), W2.astype(BF), row(b2), row(ln2_g),
      row(ln2_b))

    return out.reshape(1, L, DM)
